# Initial kernel scaffold; baseline (speedup 1.0000x reference)
import jax
import jax.numpy as jnp
from jax.experimental import pallas as pl


def kernel(user_emb, item_emb, f0, f1, f2, adj_values, adj_indices):
    raise NotImplementedError("write your pallas kernel here")


# speedup vs baseline: 4.9986x; 4.9986x over previous
import functools

import jax
import jax.numpy as jnp
from jax import lax
from jax.experimental import pallas as pl
from jax.experimental.pallas import tpu as pltpu
from jax.experimental.pallas import tpu_sc as plsc

N_TOTAL = 100000
EMB = 32
E_TOTAL = 1600000

NC = 2
NS = 16
L = 16

HALF = N_TOTAL // NC
ROWS_PER_TILE = 3128
ACC_ROWS = NS * ROWS_PER_TILE
DUMP = ACC_ROWS
HALF_PAD = ACC_ROWS
EDGES_PER_TILE = E_TOTAL // NS
K = 800
G = K // L
NCHUNK = EDGES_PER_TILE // K
ZROWS = 184


def _spmm_body(x_hbm, src_hbm, dst_hbm, val_hbm, out_hbm,
               acc_sh, gidx_v, sidx_v, val_v, rows_v, sem):
    c = lax.axis_index("c")
    s = lax.axis_index("s")

    zero16 = jnp.zeros((L,), jnp.float32)

    def _zfill(r, _):
        rows_v[r, pl.ds(0, L)] = zero16
        rows_v[r, pl.ds(L, L)] = zero16
        return 0

    lax.fori_loop(0, K, _zfill, 0)
    zbase = s * ROWS_PER_TILE
    for j in range(ROWS_PER_TILE // K):
        pltpu.sync_copy(rows_v, acc_sh.at[pl.ds(zbase + j * K, K)])
    ztail = ROWS_PER_TILE % K
    if ztail:
        pltpu.sync_copy(rows_v.at[pl.ds(0, ztail)],
                        acc_sh.at[pl.ds(zbase + (ROWS_PER_TILE // K) * K, ztail)])
    plsc.subcore_barrier()

    def _lane_bcast(v, e):
        idx = jnp.full((L, 1), e, jnp.int32)
        dnums = lax.GatherDimensionNumbers(
            offset_dims=(), collapsed_slice_dims=(0,), start_index_map=(0,))
        return lax.gather(v, idx, dnums, (1,),
                          mode=lax.GatherScatterMode.PROMISE_IN_BOUNDS)

    half_splat = jnp.full((L,), HALF, jnp.int32)
    dump_splat = jnp.full((L,), DUMP, jnp.int32)
    zero_splat = jnp.full((L,), 0, jnp.int32)

    def _chunk(i, _):
        eb = s * EDGES_PER_TILE + i * K
        pltpu.sync_copy(src_hbm.at[pl.ds(eb, K)], gidx_v)
        pltpu.sync_copy(dst_hbm.at[pl.ds(eb, K)], sidx_v)
        pltpu.sync_copy(val_hbm.at[pl.ds(eb, K)], val_v)
        pltpu.async_copy(x_hbm.at[gidx_v], rows_v, sem).wait()

        def _group(g, _):
            e0 = g * L
            v16 = val_v[pl.ds(e0, L)]
            for e in range(L):
                b = _lane_bcast(v16, e)
                r = e0 + e
                rows_v[r, pl.ds(0, L)] = rows_v[r, pl.ds(0, L)] * b
                rows_v[r, pl.ds(L, L)] = rows_v[r, pl.ds(L, L)] * b
            d16 = sidx_v[pl.ds(e0, L)]
            rel = d16 - c * HALF
            inb = (rel >= zero_splat) & (rel < half_splat)
            sidx_v[pl.ds(e0, L)] = jnp.where(inb, rel, dump_splat)
            return 0

        lax.fori_loop(0, G, _group, 0)
        pltpu.sync_copy(rows_v, acc_sh.at[sidx_v], add=True)
        return 0

    lax.fori_loop(0, NCHUNK, _chunk, 0)
    plsc.subcore_barrier()

    r0 = s * ROWS_PER_TILE
    pltpu.sync_copy(acc_sh.at[pl.ds(r0, ROWS_PER_TILE)],
                    out_hbm.at[pl.ds(c * HALF_PAD + r0, ROWS_PER_TILE)])


_spmm = pl.kernel(
    _spmm_body,
    out_type=jax.ShapeDtypeStruct((NC * HALF_PAD, EMB), jnp.float32),
    mesh=plsc.VectorSubcoreMesh(core_axis_name="c", subcore_axis_name="s",
                                num_cores=NC, num_subcores=NS),
    compiler_params=pltpu.CompilerParams(use_tc_tiling_on_sc=False),
    scratch_types=[
        pltpu.VMEM_SHARED((ACC_ROWS + 8, EMB), jnp.float32),
        pltpu.VMEM((K,), jnp.int32),
        pltpu.VMEM((K,), jnp.int32),
        pltpu.VMEM((K,), jnp.float32),
        pltpu.VMEM((K, EMB), jnp.float32),
        pltpu.SemaphoreType.DMA,
    ],
)


def _dense_body(s_ref, x_ref, w_ref, o_ref):
    z = 0.8 * s_ref[...] + 0.2 * x_ref[...]
    o_ref[...] = jax.nn.sigmoid(jnp.matmul(z, w_ref[...]))


_DBLK = 2000


def _dense(s, x, w):
    grid = (N_TOTAL // _DBLK,)
    return pl.pallas_call(
        _dense_body,
        grid=grid,
        in_specs=[
            pl.BlockSpec((_DBLK, EMB), lambda i: (i, 0)),
            pl.BlockSpec((_DBLK, EMB), lambda i: (i, 0)),
            pl.BlockSpec((EMB, EMB), lambda i: (0, 0)),
        ],
        out_specs=pl.BlockSpec((_DBLK, EMB), lambda i: (i, 0)),
        out_shape=jax.ShapeDtypeStruct((N_TOTAL, EMB), jnp.float32),
    )(s, x, w)


def kernel(user_emb, item_emb, f0, f1, f2, adj_values, adj_indices):
    n_users = user_emb.shape[0]
    x = jnp.concatenate([user_emb, item_emb], axis=0)
    src = adj_indices[1]
    dst = adj_indices[0]
    outs = [x]
    for w in (f0, f1, f2):
        padded = _spmm(x, src, dst, adj_values)
        spmm = jnp.concatenate([padded[:HALF], padded[HALF_PAD:HALF_PAD + HALF]], axis=0)
        x = _dense(spmm, x, w)
        outs.append(x)
    all_emb = jnp.concatenate(outs, axis=1)
    return (all_emb[:n_users], all_emb[n_users:])


# speedup vs baseline: 5.9577x; 1.1919x over previous
import functools

import jax
import jax.numpy as jnp
from jax import lax
from jax.experimental import pallas as pl
from jax.experimental.pallas import tpu as pltpu
from jax.experimental.pallas import tpu_sc as plsc

N_TOTAL = 100000
EMB = 32
E_TOTAL = 1600000

NC = 2
NS = 16
L = 16

HALF = N_TOTAL // NC
ROWS_PER_TILE = 3128
ACC_ROWS = NS * ROWS_PER_TILE
DUMP = ACC_ROWS
HALF_PAD = ACC_ROWS
EDGES_PER_TILE = E_TOTAL // NS
K = 400
G = K // L
NCHUNK = EDGES_PER_TILE // K


def _spmm_body(x_hbm, src_hbm, dst_hbm, val_hbm, out_hbm,
               acc_sh, gidx_v, dsti_v, val_v, sidx_v, rows_v,
               sem_i, sem_g, sem_s):
    c = lax.axis_index("c")
    s = lax.axis_index("s")

    zero16 = jnp.zeros((L,), jnp.float32)

    def _zfill(r, _):
        rows_v[0, r, pl.ds(0, L)] = zero16
        rows_v[0, r, pl.ds(L, L)] = zero16
        return 0

    lax.fori_loop(0, K, _zfill, 0)
    zbase = s * ROWS_PER_TILE
    for j in range(ROWS_PER_TILE // K):
        pltpu.sync_copy(rows_v.at[0], acc_sh.at[pl.ds(zbase + j * K, K)])
    ztail = ROWS_PER_TILE % K
    if ztail:
        pltpu.sync_copy(rows_v.at[0, pl.ds(0, ztail)],
                        acc_sh.at[pl.ds(zbase + (ROWS_PER_TILE // K) * K, ztail)])
    plsc.subcore_barrier()

    def _lane_bcast(v, e):
        idx = jnp.full((L, 1), e, jnp.int32)
        dnums = lax.GatherDimensionNumbers(
            offset_dims=(), collapsed_slice_dims=(0,), start_index_map=(0,))
        return lax.gather(v, idx, dnums, (1,),
                          mode=lax.GatherScatterMode.PROMISE_IN_BOUNDS)

    half_splat = jnp.full((L,), HALF, jnp.int32)
    dump_splat = jnp.full((L,), DUMP, jnp.int32)
    zero_splat = jnp.full((L,), 0, jnp.int32)
    ebase = s * EDGES_PER_TILE

    def _issue_idx(i, m):
        eb = ebase + i * K
        pltpu.async_copy(src_hbm.at[pl.ds(eb, K)], gidx_v.at[m], sem_i)
        pltpu.async_copy(dst_hbm.at[pl.ds(eb, K)], dsti_v.at[m], sem_i)
        pltpu.async_copy(val_hbm.at[pl.ds(eb, K)], val_v.at[m], sem_i)

    def _wait_idx():
        pltpu.make_async_copy(src_hbm.at[pl.ds(0, K)], gidx_v.at[0], sem_i).wait()
        pltpu.make_async_copy(dst_hbm.at[pl.ds(0, K)], dsti_v.at[0], sem_i).wait()
        pltpu.make_async_copy(val_hbm.at[pl.ds(0, K)], val_v.at[0], sem_i).wait()

    def _issue_gather(m):
        pltpu.async_copy(x_hbm.at[gidx_v.at[m]], rows_v.at[m], sem_g)

    def _wait_gather():
        pltpu.make_async_copy(x_hbm.at[gidx_v.at[0]], rows_v.at[0], sem_g).wait()

    def _issue_scatter(m):
        pltpu.async_copy(rows_v.at[m], acc_sh.at[sidx_v.at[m]], sem_s, add=True)

    def _wait_scatter():
        pltpu.make_async_copy(rows_v.at[0], acc_sh.at[sidx_v.at[0]], sem_s).wait()

    def _chunk(i, m):
        mb = 1 - m
        _wait_gather()

        def _group(g, _):
            e0 = g * L
            v16 = val_v[m, pl.ds(e0, L)]
            for e in range(L):
                bc = _lane_bcast(v16, e)
                r = e0 + e
                rows_v[m, r, pl.ds(0, L)] = rows_v[m, r, pl.ds(0, L)] * bc
                rows_v[m, r, pl.ds(L, L)] = rows_v[m, r, pl.ds(L, L)] * bc
            d16 = dsti_v[m, pl.ds(e0, L)]
            rel = d16 - c * HALF
            inb = (rel >= zero_splat) & (rel < half_splat)
            sidx_v[m, pl.ds(e0, L)] = jnp.where(inb, rel, dump_splat)
            return 0

        lax.fori_loop(0, G, _group, 0)

        @pl.when(i >= 1)
        def _():
            _wait_scatter()

        @pl.when(i + 1 < NCHUNK)
        def _():
            _wait_idx()
            _issue_gather(mb)

        _issue_scatter(m)

        @pl.when(i + 2 < NCHUNK)
        def _():
            _issue_idx(i + 2, m)

    _issue_idx(0, 0)
    _issue_idx(1, 1)
    _wait_idx()
    _issue_gather(0)

    def _pair(t, _):
        _chunk(2 * t, 0)
        _chunk(2 * t + 1, 1)
        return 0

    lax.fori_loop(0, NCHUNK // 2, _pair, 0)
    _wait_scatter()
    plsc.subcore_barrier()

    r0 = s * ROWS_PER_TILE
    pltpu.sync_copy(acc_sh.at[pl.ds(r0, ROWS_PER_TILE)],
                    out_hbm.at[pl.ds(c * HALF_PAD + r0, ROWS_PER_TILE)])


_spmm = pl.kernel(
    _spmm_body,
    out_type=jax.ShapeDtypeStruct((NC * HALF_PAD, EMB), jnp.float32),
    mesh=plsc.VectorSubcoreMesh(core_axis_name="c", subcore_axis_name="s",
                                num_cores=NC, num_subcores=NS),
    compiler_params=pltpu.CompilerParams(use_tc_tiling_on_sc=False),
    scratch_types=[
        pltpu.VMEM_SHARED((ACC_ROWS + 8, EMB), jnp.float32),
        pltpu.VMEM((2, K), jnp.int32),
        pltpu.VMEM((2, K), jnp.int32),
        pltpu.VMEM((2, K), jnp.float32),
        pltpu.VMEM((2, K), jnp.int32),
        pltpu.VMEM((2, K, EMB), jnp.float32),
        pltpu.SemaphoreType.DMA,
        pltpu.SemaphoreType.DMA,
        pltpu.SemaphoreType.DMA,
    ],
)


def _dense_body(s_ref, x_ref, w_ref, o_ref):
    z = 0.8 * s_ref[...] + 0.2 * x_ref[...]
    o_ref[...] = jax.nn.sigmoid(jnp.matmul(z, w_ref[...]))


_DBLK = 2000


def _dense(s, x, w):
    grid = (N_TOTAL // _DBLK,)
    return pl.pallas_call(
        _dense_body,
        grid=grid,
        in_specs=[
            pl.BlockSpec((_DBLK, EMB), lambda i: (i, 0)),
            pl.BlockSpec((_DBLK, EMB), lambda i: (i, 0)),
            pl.BlockSpec((EMB, EMB), lambda i: (0, 0)),
        ],
        out_specs=pl.BlockSpec((_DBLK, EMB), lambda i: (i, 0)),
        out_shape=jax.ShapeDtypeStruct((N_TOTAL, EMB), jnp.float32),
    )(s, x, w)


def kernel(user_emb, item_emb, f0, f1, f2, adj_values, adj_indices):
    n_users = user_emb.shape[0]
    x = jnp.concatenate([user_emb, item_emb], axis=0)
    src = adj_indices[1]
    dst = adj_indices[0]
    outs = [x]
    for w in (f0, f1, f2):
        padded = _spmm(x, src, dst, adj_values)
        spmm = jnp.concatenate([padded[:HALF], padded[HALF_PAD:HALF_PAD + HALF]], axis=0)
        x = _dense(spmm, x, w)
        outs.append(x)
    all_emb = jnp.concatenate(outs, axis=1)
    return (all_emb[:n_users], all_emb[n_users:])


# speedup vs baseline: 10.5578x; 1.7721x over previous
import functools

import jax
import jax.numpy as jnp
from jax import lax
from jax.experimental import pallas as pl
from jax.experimental.pallas import tpu as pltpu
from jax.experimental.pallas import tpu_sc as plsc

N_TOTAL = 100000
EMB = 32
E_TOTAL = 1600000

NC = 2
NS = 16
L = 16
COLS = EMB // NC

ROWS_PER_TILE = 6256
ACC_ROWS = NS * ROWS_PER_TILE
EDGES_PER_TILE = E_TOTAL // NS
K = 400
G = K // L
NCHUNK = EDGES_PER_TILE // K


def _spmm_body(x_hbm, src_hbm, dst_hbm, val_hbm, out_hbm,
               acc_sh, gidx_v, dsti_v, val_v, sidx_v, rows_v,
               sem_i, sem_g, sem_s):
    c = lax.axis_index("c")
    s = lax.axis_index("s")

    zero16 = jnp.zeros((L,), jnp.float32)

    def _zfill(r, _):
        rows_v[0, r, pl.ds(0, L)] = zero16
        return 0

    lax.fori_loop(0, K, _zfill, 0)
    zbase = s * ROWS_PER_TILE
    for j in range(ROWS_PER_TILE // K):
        pltpu.sync_copy(rows_v.at[0], acc_sh.at[pl.ds(zbase + j * K, K)])
    ztail = ROWS_PER_TILE % K
    if ztail:
        pltpu.sync_copy(rows_v.at[0, pl.ds(0, ztail)],
                        acc_sh.at[pl.ds(zbase + (ROWS_PER_TILE // K) * K, ztail)])
    plsc.subcore_barrier()

    def _lane_bcast(v, e):
        idx = jnp.full((L, 1), e, jnp.int32)
        dnums = lax.GatherDimensionNumbers(
            offset_dims=(), collapsed_slice_dims=(0,), start_index_map=(0,))
        return lax.gather(v, idx, dnums, (1,),
                          mode=lax.GatherScatterMode.PROMISE_IN_BOUNDS)

    ebase = s * EDGES_PER_TILE

    def _issue_idx(i, m):
        eb = ebase + i * K
        pltpu.async_copy(src_hbm.at[pl.ds(eb, K)], gidx_v.at[m], sem_i)
        pltpu.async_copy(dst_hbm.at[pl.ds(eb, K)], dsti_v.at[m], sem_i)
        pltpu.async_copy(val_hbm.at[pl.ds(eb, K)], val_v.at[m], sem_i)

    def _wait_idx():
        pltpu.make_async_copy(src_hbm.at[pl.ds(0, K)], gidx_v.at[0], sem_i).wait()
        pltpu.make_async_copy(dst_hbm.at[pl.ds(0, K)], dsti_v.at[0], sem_i).wait()
        pltpu.make_async_copy(val_hbm.at[pl.ds(0, K)], val_v.at[0], sem_i).wait()

    def _issue_gather(m):
        @pl.when(c == 0)
        def _():
            pltpu.async_copy(x_hbm.at[0].at[gidx_v.at[m]], rows_v.at[m], sem_g)

        @pl.when(c == 1)
        def _():
            pltpu.async_copy(x_hbm.at[1].at[gidx_v.at[m]], rows_v.at[m], sem_g)

    def _wait_gather():
        pltpu.make_async_copy(x_hbm.at[0].at[gidx_v.at[0]], rows_v.at[0],
                              sem_g).wait()

    def _issue_scatter(m):
        pltpu.async_copy(rows_v.at[m], acc_sh.at[sidx_v.at[m]], sem_s, add=True)

    def _wait_scatter():
        pltpu.make_async_copy(rows_v.at[0], acc_sh.at[sidx_v.at[0]], sem_s).wait()

    def _chunk(i, m):
        mb = 1 - m
        _wait_gather()

        def _group(g, _):
            e0 = g * L
            v16 = val_v[m, pl.ds(e0, L)]
            for e in range(L):
                bc = _lane_bcast(v16, e)
                r = e0 + e
                rows_v[m, r, pl.ds(0, L)] = rows_v[m, r, pl.ds(0, L)] * bc
            sidx_v[m, pl.ds(e0, L)] = dsti_v[m, pl.ds(e0, L)]
            return 0

        lax.fori_loop(0, G, _group, 0)

        @pl.when(i >= 1)
        def _():
            _wait_scatter()

        @pl.when(i + 1 < NCHUNK)
        def _():
            _wait_idx()
            _issue_gather(mb)

        _issue_scatter(m)

        @pl.when(i + 2 < NCHUNK)
        def _():
            _issue_idx(i + 2, m)

    _issue_idx(0, 0)
    _issue_idx(1, 1)
    _wait_idx()
    _issue_gather(0)

    def _pair(t, _):
        _chunk(2 * t, 0)
        _chunk(2 * t + 1, 1)
        return 0

    lax.fori_loop(0, NCHUNK // 2, _pair, 0)
    _wait_scatter()
    plsc.subcore_barrier()

    r0 = s * ROWS_PER_TILE

    @pl.when(c == 0)
    def _():
        pltpu.sync_copy(acc_sh.at[pl.ds(r0, ROWS_PER_TILE)],
                        out_hbm.at[0].at[pl.ds(r0, ROWS_PER_TILE)])

    @pl.when(c == 1)
    def _():
        pltpu.sync_copy(acc_sh.at[pl.ds(r0, ROWS_PER_TILE)],
                        out_hbm.at[1].at[pl.ds(r0, ROWS_PER_TILE)])


_spmm = pl.kernel(
    _spmm_body,
    out_type=jax.ShapeDtypeStruct((NC, ACC_ROWS, COLS), jnp.float32),
    mesh=plsc.VectorSubcoreMesh(core_axis_name="c", subcore_axis_name="s",
                                num_cores=NC, num_subcores=NS),
    compiler_params=pltpu.CompilerParams(use_tc_tiling_on_sc=False),
    scratch_types=[
        pltpu.VMEM_SHARED((ACC_ROWS, COLS), jnp.float32),
        pltpu.VMEM((2, K), jnp.int32),
        pltpu.VMEM((2, K), jnp.int32),
        pltpu.VMEM((2, K), jnp.float32),
        pltpu.VMEM((2, K), jnp.int32),
        pltpu.VMEM((2, K, COLS), jnp.float32),
        pltpu.SemaphoreType.DMA,
        pltpu.SemaphoreType.DMA,
        pltpu.SemaphoreType.DMA,
    ],
)


def _dense_body(s_ref, x_ref, w_ref, o_ref):
    z = 0.8 * s_ref[...] + 0.2 * x_ref[...]
    o_ref[...] = jax.nn.sigmoid(jnp.matmul(z, w_ref[...]))


_DBLK = 2000


def _dense(s, x, w):
    grid = (N_TOTAL // _DBLK,)
    return pl.pallas_call(
        _dense_body,
        grid=grid,
        in_specs=[
            pl.BlockSpec((_DBLK, EMB), lambda i: (i, 0)),
            pl.BlockSpec((_DBLK, EMB), lambda i: (i, 0)),
            pl.BlockSpec((EMB, EMB), lambda i: (0, 0)),
        ],
        out_specs=pl.BlockSpec((_DBLK, EMB), lambda i: (i, 0)),
        out_shape=jax.ShapeDtypeStruct((N_TOTAL, EMB), jnp.float32),
    )(s, x, w)


def kernel(user_emb, item_emb, f0, f1, f2, adj_values, adj_indices):
    n_users = user_emb.shape[0]
    x = jnp.concatenate([user_emb, item_emb], axis=0)
    src = adj_indices[1]
    dst = adj_indices[0]
    outs = [x]
    for w in (f0, f1, f2):
        x_split = x.reshape(N_TOTAL, NC, COLS).transpose(1, 0, 2)
        padded = _spmm(x_split, src, dst, adj_values)
        spmm = jnp.concatenate([padded[0, :N_TOTAL], padded[1, :N_TOTAL]], axis=1)
        x = _dense(spmm, x, w)
        outs.append(x)
    all_emb = jnp.concatenate(outs, axis=1)
    return (all_emb[:n_users], all_emb[n_users:])


# speedup vs baseline: 11.1885x; 1.0597x over previous
import functools

import jax
import jax.numpy as jnp
from jax import lax
from jax.experimental import pallas as pl
from jax.experimental.pallas import tpu as pltpu
from jax.experimental.pallas import tpu_sc as plsc

N_TOTAL = 100000
EMB = 32
E_TOTAL = 1600000

NC = 2
NS = 16
L = 16
COLS = EMB // NC

ROWS_PER_TILE = 6256
ACC_ROWS = NS * ROWS_PER_TILE
EDGES_PER_TILE = E_TOTAL // NS
K = 400
G = K // L
NCHUNK = EDGES_PER_TILE // K


def _spmm_body(x_hbm, src_hbm, dst_hbm, val_hbm, out_hbm,
               acc_sh, gidx_v, dsti_v, val_v, sidx_v, rows_v,
               sem_i, sem_g, sem_s):
    c = lax.axis_index("c")
    s = lax.axis_index("s")

    zero16 = jnp.zeros((L,), jnp.float32)

    def _zfill(r, _):
        rows_v[0, r, pl.ds(0, L)] = zero16
        return 0

    lax.fori_loop(0, K, _zfill, 0)
    zbase = s * ROWS_PER_TILE
    for j in range(ROWS_PER_TILE // K):
        pltpu.sync_copy(rows_v.at[0], acc_sh.at[pl.ds(zbase + j * K, K)])
    ztail = ROWS_PER_TILE % K
    if ztail:
        pltpu.sync_copy(rows_v.at[0, pl.ds(0, ztail)],
                        acc_sh.at[pl.ds(zbase + (ROWS_PER_TILE // K) * K, ztail)])
    plsc.subcore_barrier()

    def _lane_bcast(v, e):
        idx = jnp.full((L, 1), e, jnp.int32)
        dnums = lax.GatherDimensionNumbers(
            offset_dims=(), collapsed_slice_dims=(0,), start_index_map=(0,))
        return lax.gather(v, idx, dnums, (1,),
                          mode=lax.GatherScatterMode.PROMISE_IN_BOUNDS)

    ebase = s * EDGES_PER_TILE

    def _issue_idx(i, m):
        eb = ebase + i * K
        pltpu.async_copy(src_hbm.at[pl.ds(eb, K)], gidx_v.at[m], sem_i)
        pltpu.async_copy(dst_hbm.at[pl.ds(eb, K)], dsti_v.at[m], sem_i)
        pltpu.async_copy(val_hbm.at[pl.ds(eb, K)], val_v.at[m], sem_i)

    def _wait_idx():
        pltpu.make_async_copy(src_hbm.at[pl.ds(0, K)], gidx_v.at[0], sem_i).wait()
        pltpu.make_async_copy(dst_hbm.at[pl.ds(0, K)], dsti_v.at[0], sem_i).wait()
        pltpu.make_async_copy(val_hbm.at[pl.ds(0, K)], val_v.at[0], sem_i).wait()

    def _issue_gather(m):
        @pl.when(c == 0)
        def _():
            pltpu.async_copy(x_hbm.at[0].at[gidx_v.at[m]], rows_v.at[m], sem_g)

        @pl.when(c == 1)
        def _():
            pltpu.async_copy(x_hbm.at[1].at[gidx_v.at[m]], rows_v.at[m], sem_g)

    def _wait_gather():
        pltpu.make_async_copy(x_hbm.at[0].at[gidx_v.at[0]], rows_v.at[0],
                              sem_g).wait()

    def _issue_scatter(m):
        pltpu.async_copy(rows_v.at[m], acc_sh.at[sidx_v.at[m]], sem_s, add=True)

    def _wait_scatter():
        pltpu.make_async_copy(rows_v.at[0], acc_sh.at[sidx_v.at[0]], sem_s).wait()

    def _chunk(i, m):
        mb = 1 - m
        _wait_gather()

        def _group(g, _):
            e0 = g * L
            v16 = val_v[m, pl.ds(e0, L)]
            for e in range(L):
                bc = _lane_bcast(v16, e)
                r = e0 + e
                rows_v[m, r, pl.ds(0, L)] = rows_v[m, r, pl.ds(0, L)] * bc
            sidx_v[m, pl.ds(e0, L)] = dsti_v[m, pl.ds(e0, L)]
            return 0

        lax.fori_loop(0, G, _group, 0)

        @pl.when(i >= 1)
        def _():
            _wait_scatter()

        @pl.when(i + 1 < NCHUNK)
        def _():
            _wait_idx()
            _issue_gather(mb)

        _issue_scatter(m)

        @pl.when(i + 2 < NCHUNK)
        def _():
            _issue_idx(i + 2, m)

    _issue_idx(0, 0)
    _issue_idx(1, 1)
    _wait_idx()
    _issue_gather(0)

    def _pair(t, _):
        _chunk(2 * t, 0)
        _chunk(2 * t + 1, 1)
        return 0

    lax.fori_loop(0, NCHUNK // 2, _pair, 0)
    _wait_scatter()
    plsc.subcore_barrier()

    r0 = s * ROWS_PER_TILE

    @pl.when(c == 0)
    def _():
        pltpu.sync_copy(acc_sh.at[pl.ds(r0, ROWS_PER_TILE)],
                        out_hbm.at[0].at[pl.ds(r0, ROWS_PER_TILE)])

    @pl.when(c == 1)
    def _():
        pltpu.sync_copy(acc_sh.at[pl.ds(r0, ROWS_PER_TILE)],
                        out_hbm.at[1].at[pl.ds(r0, ROWS_PER_TILE)])


_spmm = pl.kernel(
    _spmm_body,
    out_type=jax.ShapeDtypeStruct((NC, ACC_ROWS, COLS), jnp.float32),
    mesh=plsc.VectorSubcoreMesh(core_axis_name="c", subcore_axis_name="s",
                                num_cores=NC, num_subcores=NS),
    compiler_params=pltpu.CompilerParams(use_tc_tiling_on_sc=False),
    scratch_types=[
        pltpu.VMEM_SHARED((ACC_ROWS, COLS), jnp.float32),
        pltpu.VMEM((2, K), jnp.int32),
        pltpu.VMEM((2, K), jnp.int32),
        pltpu.VMEM((2, K), jnp.float32),
        pltpu.VMEM((2, K), jnp.int32),
        pltpu.VMEM((2, K, COLS), jnp.float32),
        pltpu.SemaphoreType.DMA,
        pltpu.SemaphoreType.DMA,
        pltpu.SemaphoreType.DMA,
    ],
)


def _dense_body(s_ref, x_ref, w_ref, o_ref):
    s_full = jnp.concatenate([s_ref[0], s_ref[1]], axis=1)
    x_full = jnp.concatenate([x_ref[0], x_ref[1]], axis=1)
    z = 0.8 * s_full + 0.2 * x_full
    y = jax.nn.sigmoid(jnp.matmul(z, w_ref[...]))
    o_ref[0] = y[:, :COLS]
    o_ref[1] = y[:, COLS:]


_DBLK = 2000


def _dense(s_padded, x_split, w):
    grid = (N_TOTAL // _DBLK,)
    return pl.pallas_call(
        _dense_body,
        grid=grid,
        in_specs=[
            pl.BlockSpec((NC, _DBLK, COLS), lambda i: (0, i, 0)),
            pl.BlockSpec((NC, _DBLK, COLS), lambda i: (0, i, 0)),
            pl.BlockSpec((EMB, EMB), lambda i: (0, 0)),
        ],
        out_specs=pl.BlockSpec((NC, _DBLK, COLS), lambda i: (0, i, 0)),
        out_shape=jax.ShapeDtypeStruct((NC, N_TOTAL, COLS), jnp.float32),
    )(s_padded, x_split, w)


def kernel(user_emb, item_emb, f0, f1, f2, adj_values, adj_indices):
    n_users = user_emb.shape[0]
    x0 = jnp.concatenate([user_emb, item_emb], axis=0)
    src = adj_indices[1]
    dst = adj_indices[0]
    xs = x0.reshape(N_TOTAL, NC, COLS).transpose(1, 0, 2)
    outs = [x0]
    for w in (f0, f1, f2):
        padded = _spmm(xs, src, dst, adj_values)
        xs = _dense(padded, xs, w)
        outs.append(jnp.concatenate([xs[0], xs[1]], axis=1))
    all_emb = jnp.concatenate(outs, axis=1)
    return (all_emb[:n_users], all_emb[n_users:])
